# baseline (speedup 1.0000x reference)
import functools

import jax
import jax.numpy as jnp
from jax import lax
from jax.experimental import pallas as pl
from jax.experimental.pallas import tpu as pltpu
from jax.experimental.pallas import tpu_sc as plsc

_QK_DIM = 512
_P = 256
_TOPK = 8
_SCALE = _QK_DIM ** (-0.5)

_BB = 8
_NG = _P // 16
_NW = 32
_CHUNK = 128


def _logits_kernel(q_ref, k_ref, wt_ref, b_ref, out_ref):
    wt = wt_ref[...]
    brow = b_ref[...]

    row = jax.lax.broadcasted_iota(jnp.int32, (_P, _P), 0)
    col = jax.lax.broadcasted_iota(jnp.int32, (_P, _P), 1)

    for bi in range(_BB):
        qb = q_ref[bi]
        kb = k_ref[bi]
        qh = jnp.dot(qb, wt, preferred_element_type=jnp.float32) + brow
        kh = jnp.dot(kb, wt, preferred_element_type=jnp.float32) + brow
        logits = jax.lax.dot_general(
            qh * _SCALE, kh, (((1,), (1,)), ((), ())),
            preferred_element_type=jnp.float32)
        out_ref[bi] = jnp.where(row == col, 1.0, logits)


def _compute_logits(query, key, W, b):
    batch = query.shape[0]
    return pl.pallas_call(
        _logits_kernel,
        grid=(batch // _BB,),
        in_specs=[
            pl.BlockSpec((_BB, _P, _QK_DIM), lambda i: (i, 0, 0)),
            pl.BlockSpec((_BB, _P, _QK_DIM), lambda i: (i, 0, 0)),
            pl.BlockSpec((_QK_DIM, _QK_DIM), lambda i: (0, 0)),
            pl.BlockSpec((1, _QK_DIM), lambda i: (0, 0)),
        ],
        out_specs=pl.BlockSpec((_BB, _P, _P), lambda i: (i, 0, 0)),
        out_shape=jax.ShapeDtypeStruct((batch, _P, _P), jnp.float32),
        compiler_params=pltpu.CompilerParams(
            dimension_semantics=("parallel",),
        ),
    )(query, key, W.T, b.reshape(1, _QK_DIM))


def _lane():
    return lax.iota(jnp.int32, 16)


def _lane_gather(x, idx):
    return lax.gather(
        x, idx.reshape(16, 1),
        lax.GatherDimensionNumbers(
            offset_dims=(), collapsed_slice_dims=(0,), start_index_map=(0,)),
        slice_sizes=(1,),
        mode=lax.GatherScatterMode.PROMISE_IN_BOUNDS)


def _merge8(av, ai, bv, bi):
    lane = _lane()
    mask8 = lane < 8
    perm8 = (lane + 8) % 16
    cv = jnp.where(mask8, av, _lane_gather(bv, perm8))
    ci = jnp.where(mask8, ai, _lane_gather(bi, perm8))
    return plsc.sort_key_val(cv, ci, descending=True)


def _row_top8(load_group):
    pairs = []
    for g in range(_NG):
        xv = load_group(g)
        iv = _lane() + (g * 16)
        pairs.append(plsc.sort_key_val(xv, iv, descending=True))
    while len(pairs) > 1:
        pairs = [_merge8(*pairs[j], *pairs[j + 1])
                 for j in range(0, len(pairs), 2)]
    return pairs[0]


def _softmax16(v):
    lane = _lane()
    m = _lane_gather(v, lane * 0)
    e = jnp.where(lane < 8, jnp.exp(v - m), 0.0)
    return e / jnp.sum(e)


def _make_sc_topk(n_rows):
    rows_per_w = n_rows // _NW
    n_chunk = rows_per_w // _CHUNK
    mesh = plsc.VectorSubcoreMesh(core_axis_name="c", subcore_axis_name="s")

    @functools.partial(
        pl.kernel, mesh=mesh,
        out_type=[
            jax.ShapeDtypeStruct((n_rows, 16), jnp.float32),
            jax.ShapeDtypeStruct((n_rows, 16), jnp.int32),
        ],
        scratch_types=[
            pltpu.VMEM((_CHUNK, _P), jnp.float32),
            pltpu.VMEM((_CHUNK, 16), jnp.float32),
            pltpu.VMEM((_CHUNK, 16), jnp.int32),
        ],
        compiler_params=pltpu.CompilerParams(needs_layout_passes=False),
    )
    def sc_topk(x_hbm, w_hbm, i_hbm, xin, wout, iout):
        wid = lax.axis_index("s") * 2 + lax.axis_index("c")
        base = wid * rows_per_w

        def chunk_body(c, _):
            row0 = base + c * _CHUNK
            pltpu.sync_copy(x_hbm.at[pl.ds(row0, _CHUNK)], xin)

            def row_body(r, _):
                v, i = _row_top8(lambda g: xin[r, pl.ds(g * 16, 16)])
                wout[r, :] = _softmax16(v)
                iout[r, :] = i
                return 0

            lax.fori_loop(0, _CHUNK, row_body, 0, unroll=2)
            pltpu.sync_copy(wout, w_hbm.at[pl.ds(row0, _CHUNK)])
            pltpu.sync_copy(iout, i_hbm.at[pl.ds(row0, _CHUNK)])
            return 0

        lax.fori_loop(0, n_chunk, chunk_body, 0)

    return sc_topk


_NCHUNKS = 4


@jax.jit
def kernel(query, key, W, b):
    batch = query.shape[0]
    cb = batch // _NCHUNKS
    sc_topk = _make_sc_topk(cb * _P)
    ws, is_ = [], []
    for c in range(_NCHUNKS):
        sl = slice(c * cb, (c + 1) * cb)
        logits = _compute_logits(query[sl], key[sl], W, b)
        w16, i16 = sc_topk(logits.reshape(cb * _P, _P))
        ws.append(w16[:, :_TOPK].reshape(cb, _P, _TOPK))
        is_.append(i16[:, :_TOPK].reshape(cb, _P, _TOPK))
    return jnp.concatenate(ws), jnp.concatenate(is_)


# speedup vs baseline: 1.2424x; 1.2424x over previous
import functools

import jax
import jax.numpy as jnp
from jax import lax
from jax.experimental import pallas as pl
from jax.experimental.pallas import tpu as pltpu
from jax.experimental.pallas import tpu_sc as plsc

_QK_DIM = 512
_P = 256
_TOPK = 8
_SCALE = _QK_DIM ** (-0.5)

_BB = 8
_NG = _P // 16
_NW = 32
_CHUNK = 128


def _logits_kernel(q_ref, k_ref, wt_ref, b_ref, out_ref):
    wt = wt_ref[...]
    brow = b_ref[...]

    row = jax.lax.broadcasted_iota(jnp.int32, (_P, _P), 0)
    col = jax.lax.broadcasted_iota(jnp.int32, (_P, _P), 1)

    for bi in range(_BB):
        qb = q_ref[bi]
        kb = k_ref[bi]
        qh = jnp.dot(qb, wt, preferred_element_type=jnp.float32) + brow
        kh = jnp.dot(kb, wt, preferred_element_type=jnp.float32) + brow
        logits = jax.lax.dot_general(
            qh * _SCALE, kh, (((1,), (1,)), ((), ())),
            preferred_element_type=jnp.float32)
        out_ref[bi] = jnp.where(row == col, 1.0, logits)


def _compute_logits(query, key, W, b):
    batch = query.shape[0]
    return pl.pallas_call(
        _logits_kernel,
        grid=(batch // _BB,),
        in_specs=[
            pl.BlockSpec((_BB, _P, _QK_DIM), lambda i: (i, 0, 0)),
            pl.BlockSpec((_BB, _P, _QK_DIM), lambda i: (i, 0, 0)),
            pl.BlockSpec((_QK_DIM, _QK_DIM), lambda i: (0, 0)),
            pl.BlockSpec((1, _QK_DIM), lambda i: (0, 0)),
        ],
        out_specs=pl.BlockSpec((_BB, _P, _P), lambda i: (i, 0, 0)),
        out_shape=jax.ShapeDtypeStruct((batch, _P, _P), jnp.float32),
        compiler_params=pltpu.CompilerParams(
            dimension_semantics=("parallel",),
        ),
    )(query, key, W.T, b.reshape(1, _QK_DIM))


def _lane():
    return lax.iota(jnp.int32, 16)


def _lane_gather(x, idx):
    return lax.gather(
        x, idx.reshape(16, 1),
        lax.GatherDimensionNumbers(
            offset_dims=(), collapsed_slice_dims=(0,), start_index_map=(0,)),
        slice_sizes=(1,),
        mode=lax.GatherScatterMode.PROMISE_IN_BOUNDS)


def _merge8(av, ai, bv, bi):
    lane = _lane()
    mask8 = lane < 8
    perm8 = (lane + 8) % 16
    cv = jnp.where(mask8, av, _lane_gather(bv, perm8))
    ci = jnp.where(mask8, ai, _lane_gather(bi, perm8))
    return plsc.sort_key_val(cv, ci, descending=True)


def _row_top8(load_group):
    pairs = []
    for g in range(_NG):
        xv = load_group(g)
        iv = _lane() + (g * 16)
        pairs.append(plsc.sort_key_val(xv, iv, descending=True))
    while len(pairs) > 1:
        pairs = [_merge8(*pairs[j], *pairs[j + 1])
                 for j in range(0, len(pairs), 2)]
    return pairs[0]


def _softmax16(v):
    lane = _lane()
    m = _lane_gather(v, lane * 0)
    e = jnp.where(lane < 8, jnp.exp(v - m), 0.0)
    return e / jnp.sum(e)


def _make_sc_topk(n_rows):
    rows_per_w = n_rows // _NW
    n_chunk = rows_per_w // _CHUNK
    mesh = plsc.VectorSubcoreMesh(core_axis_name="c", subcore_axis_name="s")

    @functools.partial(
        pl.kernel, mesh=mesh,
        out_type=[
            jax.ShapeDtypeStruct((n_rows, 16), jnp.float32),
            jax.ShapeDtypeStruct((n_rows, 16), jnp.int32),
        ],
        scratch_types=[
            pltpu.VMEM((_CHUNK, _P), jnp.float32),
            pltpu.VMEM((_CHUNK, _P), jnp.float32),
            pltpu.VMEM((_CHUNK, 16), jnp.float32),
            pltpu.VMEM((_CHUNK, 16), jnp.int32),
            pltpu.SemaphoreType.DMA,
            pltpu.SemaphoreType.DMA,
        ],
        compiler_params=pltpu.CompilerParams(needs_layout_passes=False),
    )
    def sc_topk(x_hbm, w_hbm, i_hbm, xin0, xin1, wout, iout, sem0, sem1):
        wid = lax.axis_index("s") * 2 + lax.axis_index("c")
        base = wid * rows_per_w
        bufs = [(xin0, sem0), (xin1, sem1)]

        handles = [None] * n_chunk
        handles[0] = pltpu.async_copy(
            x_hbm.at[pl.ds(base, _CHUNK)], xin0, sem0)
        for c in range(n_chunk):
            xin, _ = bufs[c % 2]
            if c + 1 < n_chunk:
                nbuf, nsem = bufs[(c + 1) % 2]
                handles[c + 1] = pltpu.async_copy(
                    x_hbm.at[pl.ds(base + (c + 1) * _CHUNK, _CHUNK)],
                    nbuf, nsem)
            handles[c].wait()

            def row_body(r, _, xin=xin):
                v, i = _row_top8(lambda g: xin[r, pl.ds(g * 16, 16)])
                wout[r, :] = _softmax16(v)
                iout[r, :] = i
                return 0

            lax.fori_loop(0, _CHUNK, row_body, 0, unroll=4)
            row0 = base + c * _CHUNK
            pltpu.sync_copy(wout, w_hbm.at[pl.ds(row0, _CHUNK)])
            pltpu.sync_copy(iout, i_hbm.at[pl.ds(row0, _CHUNK)])

    return sc_topk


@jax.jit
def kernel(query, key, W, b):
    batch = query.shape[0]
    logits = _compute_logits(query, key, W, b)
    n_rows = batch * _P
    w16, i16 = _make_sc_topk(n_rows)(logits.reshape(n_rows, _P))
    r_weight = w16[:, :_TOPK].reshape(batch, _P, _TOPK)
    topk_index = i16[:, :_TOPK].reshape(batch, _P, _TOPK)
    return r_weight, topk_index


# speedup vs baseline: 1.2883x; 1.0370x over previous
import functools

import jax
import jax.numpy as jnp
from jax import lax
from jax.experimental import pallas as pl
from jax.experimental.pallas import tpu as pltpu
from jax.experimental.pallas import tpu_sc as plsc

_QK_DIM = 512
_P = 256
_TOPK = 8
_SCALE = _QK_DIM ** (-0.5)

_BB = 16
_NG = _P // 16
_NW = 32
_CHUNK = 128


def _logits_kernel(q_ref, k_ref, wt_ref, b_ref, out_ref):
    wt = wt_ref[...]
    brow = b_ref[...]

    row = jax.lax.broadcasted_iota(jnp.int32, (_P, _P), 0)
    col = jax.lax.broadcasted_iota(jnp.int32, (_P, _P), 1)

    for bi in range(_BB):
        qb = q_ref[bi]
        kb = k_ref[bi]
        qh = jnp.dot(qb, wt, preferred_element_type=jnp.float32) + brow
        kh = jnp.dot(kb, wt, preferred_element_type=jnp.float32) + brow
        logits = jax.lax.dot_general(
            qh * _SCALE, kh, (((1,), (1,)), ((), ())),
            preferred_element_type=jnp.float32)
        out_ref[bi] = jnp.where(row == col, 1.0, logits)


def _compute_logits(query, key, W, b):
    batch = query.shape[0]
    return pl.pallas_call(
        _logits_kernel,
        grid=(batch // _BB,),
        in_specs=[
            pl.BlockSpec((_BB, _P, _QK_DIM), lambda i: (i, 0, 0)),
            pl.BlockSpec((_BB, _P, _QK_DIM), lambda i: (i, 0, 0)),
            pl.BlockSpec((_QK_DIM, _QK_DIM), lambda i: (0, 0)),
            pl.BlockSpec((1, _QK_DIM), lambda i: (0, 0)),
        ],
        out_specs=pl.BlockSpec((_BB, _P, _P), lambda i: (i, 0, 0)),
        out_shape=jax.ShapeDtypeStruct((batch, _P, _P), jnp.float32),
        compiler_params=pltpu.CompilerParams(
            dimension_semantics=("parallel",),
        ),
    )(query, key, W.T, b.reshape(1, _QK_DIM))


def _lane():
    return lax.iota(jnp.int32, 16)


def _lane_gather(x, idx):
    return lax.gather(
        x, idx.reshape(16, 1),
        lax.GatherDimensionNumbers(
            offset_dims=(), collapsed_slice_dims=(0,), start_index_map=(0,)),
        slice_sizes=(1,),
        mode=lax.GatherScatterMode.PROMISE_IN_BOUNDS)


def _merge8(av, ai, bv, bi):
    lane = _lane()
    mask8 = lane < 8
    perm8 = (lane + 8) % 16
    cv = jnp.where(mask8, av, _lane_gather(bv, perm8))
    ci = jnp.where(mask8, ai, _lane_gather(bi, perm8))
    return plsc.sort_key_val(cv, ci, descending=True)


def _row_top8(load_group):
    pairs = []
    for g in range(_NG):
        xv = load_group(g)
        iv = _lane() + (g * 16)
        pairs.append(plsc.sort_key_val(xv, iv, descending=True))
    while len(pairs) > 1:
        pairs = [_merge8(*pairs[j], *pairs[j + 1])
                 for j in range(0, len(pairs), 2)]
    return pairs[0]


def _softmax16(v):
    lane = _lane()
    m = _lane_gather(v, lane * 0)
    e = jnp.where(lane < 8, jnp.exp(v - m), 0.0)
    return e / jnp.sum(e)


def _make_sc_topk(n_rows):
    rows_per_w = n_rows // _NW
    n_chunk = rows_per_w // _CHUNK
    mesh = plsc.VectorSubcoreMesh(core_axis_name="c", subcore_axis_name="s")

    @functools.partial(
        pl.kernel, mesh=mesh,
        out_type=[
            jax.ShapeDtypeStruct((n_rows, 16), jnp.float32),
            jax.ShapeDtypeStruct((n_rows, 16), jnp.int32),
        ],
        scratch_types=[
            pltpu.VMEM((_CHUNK, _P), jnp.float32),
            pltpu.VMEM((_CHUNK, _P), jnp.float32),
            pltpu.VMEM((_CHUNK, 16), jnp.float32),
            pltpu.VMEM((_CHUNK, 16), jnp.float32),
            pltpu.VMEM((_CHUNK, 16), jnp.int32),
            pltpu.VMEM((_CHUNK, 16), jnp.int32),
            pltpu.SemaphoreType.DMA,
            pltpu.SemaphoreType.DMA,
            pltpu.SemaphoreType.DMA,
            pltpu.SemaphoreType.DMA,
        ],
        compiler_params=pltpu.CompilerParams(needs_layout_passes=False),
    )
    def sc_topk(x_hbm, w_hbm, i_hbm, xin0, xin1, wout0, wout1,
                iout0, iout1, sem0, sem1, osem0, osem1):
        wid = lax.axis_index("s") * 2 + lax.axis_index("c")
        base = wid * rows_per_w
        bufs = [(xin0, sem0), (xin1, sem1)]
        obufs = [(wout0, iout0, osem0), (wout1, iout1, osem1)]

        handles = [None] * n_chunk
        ohandles = [None] * n_chunk
        handles[0] = pltpu.async_copy(
            x_hbm.at[pl.ds(base, _CHUNK)], xin0, sem0)
        for c in range(n_chunk):
            xin, _ = bufs[c % 2]
            wout, iout, osem = obufs[c % 2]
            if c + 1 < n_chunk:
                nbuf, nsem = bufs[(c + 1) % 2]
                handles[c + 1] = pltpu.async_copy(
                    x_hbm.at[pl.ds(base + (c + 1) * _CHUNK, _CHUNK)],
                    nbuf, nsem)
            handles[c].wait()
            if c >= 2:
                for h in ohandles[c - 2]:
                    h.wait()

            def row_body(r, _, xin=xin, wout=wout, iout=iout):
                v, i = _row_top8(lambda g: xin[r, pl.ds(g * 16, 16)])
                wout[r, :] = _softmax16(v)
                iout[r, :] = i
                return 0

            lax.fori_loop(0, _CHUNK, row_body, 0, unroll=4)
            row0 = base + c * _CHUNK
            ohandles[c] = (
                pltpu.async_copy(wout, w_hbm.at[pl.ds(row0, _CHUNK)], osem),
                pltpu.async_copy(iout, i_hbm.at[pl.ds(row0, _CHUNK)], osem),
            )
        for c in (n_chunk - 2, n_chunk - 1):
            for h in ohandles[c]:
                h.wait()

    return sc_topk


@jax.jit
def kernel(query, key, W, b):
    batch = query.shape[0]
    logits = _compute_logits(query, key, W, b)
    n_rows = batch * _P
    w16, i16 = _make_sc_topk(n_rows)(logits.reshape(n_rows, _P))
    r_weight = w16[:, :_TOPK].reshape(batch, _P, _TOPK)
    topk_index = i16[:, :_TOPK].reshape(batch, _P, _TOPK)
    return r_weight, topk_index
